# baseline (speedup 1.0000x reference)
import numpy as np
import jax
import jax.numpy as jnp
from jax.experimental import pallas as pl
from jax.experimental.pallas import tpu as pltpu

_P = 0.1
_B, _T, _E = 32, 1024, 768
_N = _B * _T

_ROT_A = (13, 15, 26, 6)
_ROT_B = (17, 29, 16, 24)


def _host_threefry2x32(k1, k2, x0, x1):
    def rotl(x, d):
        return ((x << np.uint32(d)) | (x >> np.uint32(32 - d))).astype(np.uint32)

    def rounds(x0, x1, rots):
        for r in rots:
            x0 = (x0 + x1).astype(np.uint32)
            x1 = x0 ^ rotl(x1, r)
        return x0, x1

    ks0, ks1 = np.uint32(k1), np.uint32(k2)
    ks2 = np.uint32(ks0 ^ ks1 ^ np.uint32(0x1BD11BDA))
    x0 = (np.uint32(x0) + ks0).astype(np.uint32)
    x1 = (np.uint32(x1) + ks1).astype(np.uint32)
    x0, x1 = rounds(x0, x1, _ROT_A)
    x0 = (x0 + ks1).astype(np.uint32)
    x1 = (x1 + ks2 + np.uint32(1)).astype(np.uint32)
    x0, x1 = rounds(x0, x1, _ROT_B)
    x0 = (x0 + ks2).astype(np.uint32)
    x1 = (x1 + ks0 + np.uint32(2)).astype(np.uint32)
    x0, x1 = rounds(x0, x1, _ROT_A)
    x0 = (x0 + ks0).astype(np.uint32)
    x1 = (x1 + ks1 + np.uint32(3)).astype(np.uint32)
    x0, x1 = rounds(x0, x1, _ROT_B)
    x0 = (x0 + ks1).astype(np.uint32)
    x1 = (x1 + ks2 + np.uint32(4)).astype(np.uint32)
    x0, x1 = rounds(x0, x1, _ROT_A)
    x0 = (x0 + ks2).astype(np.uint32)
    x1 = (x1 + ks0 + np.uint32(5)).astype(np.uint32)
    return x0, x1


_K1, _K2 = (int(v[0]) for v in
            _host_threefry2x32(0, 0, np.uint32([0]), np.uint32([1])))


def _bits_from_index(idx):
    sru = jax.lax.shift_right_logical

    def rounds(x0, x1, rots):
        for r in rots:
            x0 = x0 + x1
            x1 = x0 ^ ((x1 << jnp.uint32(r)) | sru(x1, jnp.uint32(32 - r)))
        return x0, x1

    ks0 = jnp.uint32(_K1)
    ks1 = jnp.uint32(_K2)
    ks2 = jnp.uint32(_K1 ^ _K2 ^ 0x1BD11BDA)
    x0 = jnp.full(idx.shape, ks0, jnp.uint32)
    x1 = idx + ks1
    x0, x1 = rounds(x0, x1, _ROT_A)
    x0 = x0 + ks1
    x1 = x1 + (ks2 + jnp.uint32(1))
    x0, x1 = rounds(x0, x1, _ROT_B)
    x0 = x0 + ks2
    x1 = x1 + (ks0 + jnp.uint32(2))
    x0, x1 = rounds(x0, x1, _ROT_A)
    x0 = x0 + ks0
    x1 = x1 + (ks1 + jnp.uint32(3))
    x0, x1 = rounds(x0, x1, _ROT_B)
    x0 = x0 + ks1
    x1 = x1 + (ks2 + jnp.uint32(4))
    x0, x1 = rounds(x0, x1, _ROT_A)
    x0 = x0 + ks2
    x1 = x1 + (ks0 + jnp.uint32(5))
    return x0 ^ x1


def _keep_from_index(idx):
    bits = _bits_from_index(idx)
    fb = jax.lax.shift_right_logical(bits, jnp.uint32(9)) | jnp.uint32(0x3F800000)
    u = jax.lax.bitcast_convert_type(fb, jnp.float32) - jnp.float32(1.0)
    return jnp.where(u < jnp.float32(_P), jnp.float32(0.0), jnp.float32(1.0))


_MROWS, _MCOLS = 256, 128


def _mask_kernel(o_ref):
    s = jax.lax.broadcasted_iota(jnp.uint32, (_MROWS, _MCOLS), 0)
    l = jax.lax.broadcasted_iota(jnp.uint32, (_MROWS, _MCOLS), 1)
    o_ref[...] = _keep_from_index(s * jnp.uint32(_MCOLS) + l)


_BLK = 4096


def _fused_kernel(x_ref, o_ref, m_ref):
    i = pl.program_id(0)

    @pl.when(i == 0)
    def _():
        s = jax.lax.broadcasted_iota(jnp.uint32, (_MROWS, _MCOLS), 0)
        l = jax.lax.broadcasted_iota(jnp.uint32, (_MROWS, _MCOLS), 1)
        m_ref[...] = _keep_from_index(s * jnp.uint32(_MCOLS) + l)

    rows_per = _BLK // _MCOLS
    m = m_ref[pl.ds(i * rows_per, rows_per), :]
    b = jnp.reshape(
        jax.lax.broadcast_in_dim(m, (rows_per, _MCOLS, _MCOLS), (0, 2)),
        (_BLK, _MCOLS))
    r_sub = jax.lax.broadcasted_iota(jnp.int32, (_BLK, _MCOLS), 0)
    r_lane = jax.lax.broadcasted_iota(jnp.int32, (_BLK, _MCOLS), 1)
    onehot = (r_sub % _MCOLS == r_lane).astype(jnp.float32)
    mcol = jnp.sum(b * onehot, axis=1, keepdims=True)
    o_ref[...] = x_ref[...] * mcol


def kernel(X):
    Xf = X.reshape(_N, _E)
    out = pl.pallas_call(
        _fused_kernel,
        grid=(_N // _BLK,),
        in_specs=[
            pl.BlockSpec((_BLK, _E), lambda i: (i, 0)),
        ],
        out_specs=pl.BlockSpec((_BLK, _E), lambda i: (i, 0)),
        out_shape=jax.ShapeDtypeStruct((_N, _E), jnp.float32),
        scratch_shapes=[pltpu.VMEM((_MROWS, _MCOLS), jnp.float32)],
    )(Xf)
    return out.reshape(_B, _T, _E)


# speedup vs baseline: 1.0006x; 1.0006x over previous
import numpy as np
import jax
import jax.numpy as jnp
from jax.experimental import pallas as pl
from jax.experimental.pallas import tpu as pltpu

_P = 0.1
_B, _T, _E = 32, 1024, 768
_N = _B * _T

_ROT_A = (13, 15, 26, 6)
_ROT_B = (17, 29, 16, 24)


def _host_threefry2x32(k1, k2, x0, x1):
    def rotl(x, d):
        return ((x << np.uint32(d)) | (x >> np.uint32(32 - d))).astype(np.uint32)

    def rounds(x0, x1, rots):
        for r in rots:
            x0 = (x0 + x1).astype(np.uint32)
            x1 = x0 ^ rotl(x1, r)
        return x0, x1

    ks0, ks1 = np.uint32(k1), np.uint32(k2)
    ks2 = np.uint32(ks0 ^ ks1 ^ np.uint32(0x1BD11BDA))
    x0 = (np.uint32(x0) + ks0).astype(np.uint32)
    x1 = (np.uint32(x1) + ks1).astype(np.uint32)
    x0, x1 = rounds(x0, x1, _ROT_A)
    x0 = (x0 + ks1).astype(np.uint32)
    x1 = (x1 + ks2 + np.uint32(1)).astype(np.uint32)
    x0, x1 = rounds(x0, x1, _ROT_B)
    x0 = (x0 + ks2).astype(np.uint32)
    x1 = (x1 + ks0 + np.uint32(2)).astype(np.uint32)
    x0, x1 = rounds(x0, x1, _ROT_A)
    x0 = (x0 + ks0).astype(np.uint32)
    x1 = (x1 + ks1 + np.uint32(3)).astype(np.uint32)
    x0, x1 = rounds(x0, x1, _ROT_B)
    x0 = (x0 + ks1).astype(np.uint32)
    x1 = (x1 + ks2 + np.uint32(4)).astype(np.uint32)
    x0, x1 = rounds(x0, x1, _ROT_A)
    x0 = (x0 + ks2).astype(np.uint32)
    x1 = (x1 + ks0 + np.uint32(5)).astype(np.uint32)
    return x0, x1


_K1, _K2 = (int(v[0]) for v in
            _host_threefry2x32(0, 0, np.uint32([0]), np.uint32([1])))


def _bits_from_index(idx):
    sru = jax.lax.shift_right_logical

    def rounds(x0, x1, rots):
        for r in rots:
            x0 = x0 + x1
            x1 = x0 ^ ((x1 << jnp.uint32(r)) | sru(x1, jnp.uint32(32 - r)))
        return x0, x1

    ks0 = jnp.uint32(_K1)
    ks1 = jnp.uint32(_K2)
    ks2 = jnp.uint32(_K1 ^ _K2 ^ 0x1BD11BDA)
    x0 = jnp.full(idx.shape, ks0, jnp.uint32)
    x1 = idx + ks1
    x0, x1 = rounds(x0, x1, _ROT_A)
    x0 = x0 + ks1
    x1 = x1 + (ks2 + jnp.uint32(1))
    x0, x1 = rounds(x0, x1, _ROT_B)
    x0 = x0 + ks2
    x1 = x1 + (ks0 + jnp.uint32(2))
    x0, x1 = rounds(x0, x1, _ROT_A)
    x0 = x0 + ks0
    x1 = x1 + (ks1 + jnp.uint32(3))
    x0, x1 = rounds(x0, x1, _ROT_B)
    x0 = x0 + ks1
    x1 = x1 + (ks2 + jnp.uint32(4))
    x0, x1 = rounds(x0, x1, _ROT_A)
    x0 = x0 + ks2
    x1 = x1 + (ks0 + jnp.uint32(5))
    return x0 ^ x1


def _keep_from_index(idx):
    bits = _bits_from_index(idx)
    fb = jax.lax.shift_right_logical(bits, jnp.uint32(9)) | jnp.uint32(0x3F800000)
    u = jax.lax.bitcast_convert_type(fb, jnp.float32) - jnp.float32(1.0)
    return jnp.where(u < jnp.float32(_P), jnp.float32(0.0), jnp.float32(1.0))


_MROWS, _MCOLS = 256, 128

_BLK = 4096


def _fused_kernel(x_ref, o_ref, m_ref):
    i = pl.program_id(0)

    @pl.when(i == 0)
    def _():
        s = jax.lax.broadcasted_iota(jnp.uint32, (_MROWS, _MCOLS), 0)
        l = jax.lax.broadcasted_iota(jnp.uint32, (_MROWS, _MCOLS), 1)
        m_ref[...] = _keep_from_index(s * jnp.uint32(_MCOLS) + l)

    rows_per = _BLK // _MCOLS
    m = m_ref[pl.ds(i * rows_per, rows_per), :]
    b = jnp.reshape(
        jax.lax.broadcast_in_dim(m, (rows_per, _MCOLS, _MCOLS), (0, 2)),
        (_BLK, _MCOLS))
    r_sub = jax.lax.broadcasted_iota(jnp.int32, (_BLK, _MCOLS), 0)
    r_lane = jax.lax.broadcasted_iota(jnp.int32, (_BLK, _MCOLS), 1)
    onehot = (r_sub % _MCOLS == r_lane).astype(jnp.float32)
    mcol = jnp.sum(b * onehot, axis=1, keepdims=True)
    o_ref[...] = x_ref[...] * mcol


def kernel(X):
    Xf = X.reshape(_N, _E)
    out = pl.pallas_call(
        _fused_kernel,
        grid=(_N // _BLK,),
        in_specs=[
            pl.BlockSpec((_BLK, _E), lambda i: (i, 0)),
        ],
        out_specs=pl.BlockSpec((_BLK, _E), lambda i: (i, 0)),
        out_shape=jax.ShapeDtypeStruct((_N, _E), jnp.float32),
        scratch_shapes=[pltpu.VMEM((_MROWS, _MCOLS), jnp.float32)],
    )(Xf)
    return out.reshape(_B, _T, _E)
